# baseline (speedup 1.0000x reference)
import functools

import jax
import jax.numpy as jnp
from jax import lax
from jax.experimental import pallas as pl
from jax.experimental.pallas import tpu as pltpu
from jax.experimental.pallas import tpu_sc as plsc

F = 26
V = 100000
D = 16
B = 16384

_info = plsc.get_sparse_core_info()
NC, NS, L = _info.num_cores, _info.num_subcores, _info.num_lanes
J = F * D
SPC = J // NC
BPT = B // NS
NBUF = 4

_mesh = plsc.VectorSubcoreMesh(core_axis_name="c", subcore_axis_name="s")


@functools.partial(
    pl.kernel,
    out_type=jax.ShapeDtypeStruct((J, B), jnp.float32),
    mesh=_mesh,
    compiler_params=pltpu.CompilerParams(use_tc_tiling_on_sc=True),
    scratch_types=[
        [pltpu.VMEM_SHARED((V,), jnp.float32) for _ in range(NBUF)],
        [pltpu.VMEM((BPT,), jnp.int32) for _ in range(2)],
        [pltpu.VMEM((BPT,), jnp.float32) for _ in range(NBUF)],
        pltpu.SemaphoreType.DMA,
        pltpu.SemaphoreType.DMA,
        pltpu.SemaphoreType.DMA,
        pltpu.SemaphoreType.DMA,
    ],
)
def _sc_lookup(tab_hbm, idx_hbm, out_hbm, st, iv, cv, fsem, isem, gsem, wsem):
    c = lax.axis_index("c")
    s = lax.axis_index("s")
    j0 = c * SPC
    col = pl.ds(s * BPT, BPT)

    for p in range(NBUF - 1):
        @pl.when(s == p)
        def _prime_fill(p=p):
            pltpu.async_copy(tab_hbm.at[(j0 + p) >> 4, (j0 + p) & 15], st[p], fsem)

    pltpu.sync_copy(idx_hbm.at[j0 >> 4, col], iv[0])

    @pl.when(s == 0)
    def _wait_fill0():
        pltpu.make_async_copy(tab_hbm.at[j0 >> 4, j0 & 15], st[0], fsem).wait()

    plsc.subcore_barrier()

    def quad(t, carry):
        for q in range(NBUF):
            r = NBUF * t + q
            j = j0 + r
            jn = j + 1
            has_next = r + 1 < SPC
            has_far = r + 3 < SPC

            @pl.when((s == ((r + 3) & 15)) & has_far)
            def _start_fill():
                jf = j + 3
                pltpu.async_copy(tab_hbm.at[jf >> 4, jf & 15], st[(q + 3) % NBUF], fsem)

            @pl.when(has_next)
            def _start_idx():
                pltpu.async_copy(idx_hbm.at[jn >> 4, col], iv[(q + 1) % 2], isem)

            @pl.when(r >= NBUF)
            def _drain_old_write():
                pltpu.make_async_copy(cv[q], out_hbm.at[j, col], wsem).wait()

            pltpu.async_copy(st[q].at[iv[q % 2]], cv[q], gsem).wait()
            pltpu.async_copy(cv[q], out_hbm.at[j, col], wsem)

            @pl.when(has_next)
            def _wait_idx():
                pltpu.make_async_copy(idx_hbm.at[jn >> 4, col], iv[(q + 1) % 2], isem).wait()

            @pl.when((s == ((r + 1) & 15)) & has_next)
            def _wait_fill():
                pltpu.make_async_copy(tab_hbm.at[jn >> 4, jn & 15], st[(q + 1) % NBUF], fsem).wait()

            plsc.subcore_barrier()
        return carry

    lax.fori_loop(0, SPC // NBUF, quad, 0)
    for q in range(NBUF):
        pltpu.make_async_copy(cv[q], out_hbm.at[j0, col], wsem).wait()


def kernel(indices, tables):
    tab2 = jnp.transpose(tables, (0, 2, 1))
    out = _sc_lookup(tab2, indices)
    return out.T
